# baseline (speedup 1.0000x reference)
import functools

import jax
import jax.numpy as jnp
from jax import lax
from jax.experimental import pallas as pl
from jax.experimental.pallas import tpu as pltpu
from jax.experimental.pallas import tpu_sc as plsc

D = 16
KCHUNK = 512
GRP = 128


@functools.partial(jax.jit, static_argnames=("n_total",))
def _sc_embed(inp2d, sw, mask2d, aw, a0v, a1v, fkb2d, *, n_total):
    info = plsc.get_sparse_core_info()
    num_cores, num_subcores = info.num_cores, info.num_subcores
    num_workers = num_cores * num_subcores
    per_w = n_total // num_workers
    n_chunks = per_w // KCHUNK
    n_grp = KCHUNK // GRP

    mesh = plsc.VectorSubcoreMesh(core_axis_name="c", subcore_axis_name="s")

    @functools.partial(
        pl.kernel,
        mesh=mesh,
        compiler_params=pltpu.CompilerParams(use_tc_tiling_on_sc=False),
        out_type=jax.ShapeDtypeStruct((n_total, D), jnp.float32),
        scratch_types=[
            pltpu.VMEM((n_grp, GRP), jnp.int32),
            pltpu.VMEM((KCHUNK, D), jnp.float32),
            pltpu.VMEM((KCHUNK, D), jnp.float32),
            pltpu.VMEM((KCHUNK, 2 * D), jnp.float32),
            pltpu.VMEM((KCHUNK,), jnp.float32),
            pltpu.VMEM((KCHUNK, D), jnp.float32),
            pltpu.VMEM((16,), jnp.float32),
            pltpu.VMEM((16,), jnp.float32),
            pltpu.SemaphoreType.DMA,
        ],
    )
    def k(inp_hbm, sw_hbm, mask_hbm, aw_hbm, a0_hbm, a1_hbm, fkb_hbm, out_hbm,
          idx_v, sw_v, aw_v, fkb_v, msk_v, out_v, a0_v, a1_v, sem):
        wid = lax.axis_index("s") * num_cores + lax.axis_index("c")
        base = wid * per_w

        pltpu.sync_copy(a0_hbm, a0_v)
        pltpu.sync_copy(a1_hbm, a1_v)
        lane = lax.iota(jnp.int32, 16)
        ail = jnp.where(lane % 2 == 0, a0_v[...], a1_v[...])
        swap_idx = lane ^ 1
        evt_idx = (lane * 2) & 15
        lane_lt8 = lane < 8

        perm_dn = lax.GatherDimensionNumbers(
            offset_dims=(), collapsed_slice_dims=(0,), start_index_map=(0,))

        def _perm(x, idx):
            return lax.gather(x, idx[:, None], perm_dn, (1,),
                              mode=lax.GatherScatterMode.PROMISE_IN_BOUNDS)

        def chunk_body(c, _):
            off = base + c * KCHUNK
            idx_copies = [
                pltpu.async_copy(inp_hbm.at[pl.ds(off + j * GRP, GRP)],
                                 idx_v.at[j], sem)
                for j in range(n_grp)
            ]
            for cp in idx_copies:
                cp.wait()

            copies = []
            for j in range(n_grp):
                row = pl.ds(j * GRP, GRP)
                idx_j = idx_v.at[j]
                copies.append(pltpu.async_copy(sw_hbm.at[idx_j], sw_v.at[row], sem))
                copies.append(pltpu.async_copy(aw_hbm.at[idx_j], aw_v.at[row], sem))
                copies.append(pltpu.async_copy(fkb_hbm.at[idx_j], fkb_v.at[row], sem))
                copies.append(pltpu.async_copy(mask_hbm.at[idx_j], msk_v.at[row], sem))
            for cp in copies:
                cp.wait()

            def grp_body(g, _):
                sig16 = 1.0 / (1.0 + jnp.exp(-msk_v[pl.ds(g * 16, 16)]))
                base_i = g * 16
                for j in range(16):
                    i = base_i + j
                    u0 = ail * fkb_v[i, pl.ds(0, 16)]
                    u1 = ail * fkb_v[i, pl.ds(16, 16)]
                    v0 = u0 + _perm(u0, swap_idx)
                    v1 = u1 + _perm(u1, swap_idx)
                    comb = jnp.where(lane_lt8, _perm(v0, evt_idx),
                                     _perm(v1, evt_idx))
                    out_v[i] = sw_v[i] * sig16[j] + aw_v[i] + comb
                return 0

            lax.fori_loop(0, KCHUNK // 16, grp_body, 0)

            pltpu.sync_copy(out_v, out_hbm.at[pl.ds(off, KCHUNK)])
            return 0

        lax.fori_loop(0, n_chunks, chunk_body, 0)

    return k(inp2d, sw, mask2d, aw, a0v, a1v, fkb2d)


def kernel(input, sw, mask, aw, atten, from_kb):
    B, L = input.shape
    n_total = B * L
    inp2d = input.reshape(n_total).astype(jnp.int32)
    fkb2d = from_kb.reshape(from_kb.shape[0], -1)
    mask2d = mask
    a0v = jnp.full((16,), atten[0], jnp.float32)
    a1v = jnp.full((16,), atten[1], jnp.float32)
    out = _sc_embed(inp2d, sw, mask2d, aw, a0v, a1v, fkb2d, n_total=n_total)
    return out.reshape(B, L, D)
